# Initial kernel scaffold; baseline (speedup 1.0000x reference)
import jax
import jax.numpy as jnp
from jax.experimental import pallas as pl


def kernel(z, codebook):
    raise NotImplementedError("write your pallas kernel here")


# speedup vs baseline: 1.2951x; 1.2951x over previous
import jax
import jax.numpy as jnp
from jax import lax
from jax.experimental import pallas as pl
from jax.experimental.pallas import tpu as pltpu
from jax.experimental.pallas import tpu_sc as plsc

_K = 8192
_D = 256
_TM = 256


def _vq_body(z_ref, cb_ref, idx_ref, oh_ref, ppl_ref, cn_ref, acc_ref):
    step = pl.program_id(0)
    nsteps = pl.num_programs(0)

    @pl.when(step == 0)
    def _():
        cb = cb_ref[...]
        cn_col = jnp.sum(cb * cb, axis=1, keepdims=True)
        cn_ref[...] = cn_col.T
        acc_ref[...] = jnp.zeros_like(acc_ref)

    z = z_ref[...]
    zn = jnp.sum(z * z, axis=1, keepdims=True)
    dot = lax.dot_general(
        z, cb_ref[...], (((1,), (1,)), ((), ())),
        preferred_element_type=jnp.float32,
    )
    d = (zn - 2.0 * dot) + cn_ref[...]
    vmin = jnp.min(d, axis=1, keepdims=True)
    iota = lax.broadcasted_iota(jnp.int32, (_TM, _K), 1)
    idx = jnp.min(
        jnp.where(d == vmin, iota, jnp.int32(_K)), axis=1, keepdims=True
    )
    idx_ref[...] = idx
    oh = jnp.where(iota == idx, jnp.float32(1), jnp.float32(0))
    oh_ref[...] = oh
    acc_ref[...] = acc_ref[...] + jnp.sum(oh, axis=0, keepdims=True)

    @pl.when(step == nsteps - 1)
    def _():
        total = jnp.float32(nsteps * _TM)
        p = acc_ref[...] / total
        s = jnp.sum(p * jnp.log(p + 1e-10), axis=1, keepdims=True)
        ppl_ref[...] = jnp.exp(-s)


def _tc_quantize(z_flat, codebook):
    m = z_flat.shape[0]
    grid = (m // _TM,)
    return pl.pallas_call(
        _vq_body,
        grid=grid,
        in_specs=[
            pl.BlockSpec((_TM, _D), lambda i: (i, 0)),
            pl.BlockSpec((_K, _D), lambda i: (0, 0)),
        ],
        out_specs=[
            pl.BlockSpec((_TM, 1), lambda i: (i, 0)),
            pl.BlockSpec((_TM, _K), lambda i: (i, 0)),
            pl.BlockSpec((1, 1), lambda i: (0, 0)),
        ],
        out_shape=[
            jax.ShapeDtypeStruct((m, 1), jnp.int32),
            jax.ShapeDtypeStruct((m, _K), jnp.float32),
            jax.ShapeDtypeStruct((1, 1), jnp.float32),
        ],
        scratch_shapes=[
            pltpu.VMEM((1, _K), jnp.float32),
            pltpu.VMEM((1, _K), jnp.float32),
        ],
    )(z_flat, codebook)


_GATHER_W = 128


def _sc_gather(codebook, indices):
    n = indices.shape[0]
    idx2 = indices.reshape(1, n)
    mesh = plsc.VectorSubcoreMesh(core_axis_name="core", subcore_axis_name="subcore")

    @pl.kernel(
        out_type=jax.ShapeDtypeStruct((n, _D), codebook.dtype),
        mesh=mesh,
    )
    def gather_kernel(cb_hbm, i_hbm, o_hbm):
        def body(i_vmem, o_vmem):
            pltpu.sync_copy(cb_hbm.at[i_vmem.at[0]], o_vmem)

        pltpu.emit_pipeline(
            body,
            grid=(n // _GATHER_W,),
            in_specs=[pl.BlockSpec((1, _GATHER_W), index_map=lambda i: (0, i))],
            out_specs=[pl.BlockSpec((_GATHER_W, _D), index_map=lambda i: (i, 0))],
            core_axis_name=("core", "subcore"),
            dimension_semantics=(pltpu.PARALLEL,),
        )(i_hbm, o_hbm)

    return gather_kernel(codebook, idx2)


def kernel(z, codebook):
    b, n, d = z.shape
    z_flat = z.reshape(b * n, d)
    idx, min_encodings, ppl = _tc_quantize(z_flat, codebook)
    e_indices = idx.reshape(-1)
    z_quantized = _sc_gather(codebook, e_indices).reshape(b, n, d)
    return z_quantized, min_encodings, e_indices, ppl[0, 0]
